# Initial kernel scaffold; baseline (speedup 1.0000x reference)
import jax
import jax.numpy as jnp
from jax.experimental import pallas as pl


def kernel(x, edge_index, Wl1, bl1, Wr1, Wl2, bl2, Wr2, Wf1, bf1, Wf2, bf2):
    raise NotImplementedError("write your pallas kernel here")


# speedup vs baseline: 9.3595x; 9.3595x over previous
import functools

import jax
import jax.numpy as jnp
from jax import lax
from jax.experimental import pallas as pl
from jax.experimental.pallas import tpu as pltpu
from jax.experimental.pallas import tpu_sc as plsc

N_NODES = 10000
N_EDGES = 320000
D_FEAT = 128
HID = 64
EXT = HID + 16

_NC = 2
_NS = 16
_NW = _NC * _NS

_CH = 80
_EPW = N_EDGES // _NW
_NCHUNK = _EPW // _CH
_NPAD = 10240
_RPT = _NPAD // _NS
_RCOPY = 128

_BLK = 2000
_GRID = N_NODES // _BLK


def _dot_t(a, w):
    return lax.dot_general(a, w, (((1,), (1,)), ((), ())),
                           preferred_element_type=jnp.float32)


def _tc1_body(x_ref, wl_ref, wr_ref, y_ref, z_ref):
    xb = x_ref[...]
    y = _dot_t(xb, wl_ref[...])
    z = _dot_t(xb, wr_ref[...])
    y_ref[...] = jnp.concatenate(
        [y, jnp.ones((y.shape[0], EXT - HID), jnp.float32)], axis=1)
    z_ref[...] = z


def _tc1(x, Wl1, Wr1):
    return pl.pallas_call(
        _tc1_body,
        grid=(_GRID,),
        in_specs=[
            pl.BlockSpec((_BLK, D_FEAT), lambda i: (i, 0)),
            pl.BlockSpec((HID, D_FEAT), lambda i: (0, 0)),
            pl.BlockSpec((HID, D_FEAT), lambda i: (0, 0)),
        ],
        out_specs=[
            pl.BlockSpec((_BLK, EXT), lambda i: (i, 0)),
            pl.BlockSpec((_BLK, HID), lambda i: (i, 0)),
        ],
        out_shape=[
            jax.ShapeDtypeStruct((N_NODES, EXT), jnp.float32),
            jax.ShapeDtypeStruct((N_NODES, HID), jnp.float32),
        ],
    )(x, Wl1, Wr1)


def _tc2_body(aggp_ref, z1_ref, bl1_ref, wl2_ref, wr2_ref,
              y2_ref, z2_ref, recip_ref):
    a = aggp_ref[0] + aggp_ref[1]
    cnt = jnp.max(a[:, HID:EXT], axis=1, keepdims=True)
    recip = 1.0 / jnp.maximum(cnt, 1.0)
    h1 = jnp.maximum(a[:, :HID] * recip + bl1_ref[...] + z1_ref[...], 0.0)
    y2_ref[...] = _dot_t(h1, wl2_ref[...])
    z2_ref[...] = _dot_t(h1, wr2_ref[...])
    recip_ref[...] = recip


def _tc2(aggp, z1, bl1, Wl2, Wr2):
    return pl.pallas_call(
        _tc2_body,
        grid=(_GRID,),
        in_specs=[
            pl.BlockSpec((_NC, _BLK, EXT), lambda i: (0, i, 0)),
            pl.BlockSpec((_BLK, HID), lambda i: (i, 0)),
            pl.BlockSpec((1, HID), lambda i: (0, 0)),
            pl.BlockSpec((HID, HID), lambda i: (0, 0)),
            pl.BlockSpec((HID, HID), lambda i: (0, 0)),
        ],
        out_specs=[
            pl.BlockSpec((_BLK, HID), lambda i: (i, 0)),
            pl.BlockSpec((_BLK, HID), lambda i: (i, 0)),
            pl.BlockSpec((_BLK, 1), lambda i: (i, 0)),
        ],
        out_shape=[
            jax.ShapeDtypeStruct((N_NODES, HID), jnp.float32),
            jax.ShapeDtypeStruct((N_NODES, HID), jnp.float32),
            jax.ShapeDtypeStruct((N_NODES, 1), jnp.float32),
        ],
    )(aggp, z1, bl1, Wl2, Wr2)


def _tc3_body(aggp_ref, z2_ref, bl2_ref, recip_ref,
              wf1_ref, bf1_ref, wf2_ref, bf2_ref, out_ref, acc_ref):
    i = pl.program_id(0)

    @pl.when(i == 0)
    def _():
        acc_ref[...] = jnp.zeros_like(acc_ref)

    a = aggp_ref[0] + aggp_ref[1]
    h2 = jnp.maximum(a * recip_ref[...] + bl2_ref[...] + z2_ref[...], 0.0)
    acc_ref[...] += jnp.sum(h2, axis=0, keepdims=True)

    @pl.when(i == pl.num_programs(0) - 1)
    def _():
        g = acc_ref[...] * (1.0 / N_NODES)
        h = jnp.maximum(_dot_t(g, wf1_ref[...]) + bf1_ref[...], 0.0)
        o = _dot_t(h, wf2_ref[...]) + bf2_ref[...]
        m = jnp.max(o, axis=1, keepdims=True)
        e = jnp.exp(o - m)
        out_ref[...] = e / jnp.sum(e, axis=1, keepdims=True)


def _tc3(aggp, z2, bl2, recip, Wf1, bf1, Wf2, bf2):
    nout = Wf2.shape[0]
    return pl.pallas_call(
        _tc3_body,
        grid=(_GRID,),
        in_specs=[
            pl.BlockSpec((_NC, _BLK, HID), lambda i: (0, i, 0)),
            pl.BlockSpec((_BLK, HID), lambda i: (i, 0)),
            pl.BlockSpec((1, HID), lambda i: (0, 0)),
            pl.BlockSpec((_BLK, 1), lambda i: (i, 0)),
            pl.BlockSpec((HID, HID), lambda i: (0, 0)),
            pl.BlockSpec((1, HID), lambda i: (0, 0)),
            pl.BlockSpec((nout, HID), lambda i: (0, 0)),
            pl.BlockSpec((1, nout), lambda i: (0, 0)),
        ],
        out_specs=pl.BlockSpec((1, nout), lambda i: (0, 0)),
        out_shape=jax.ShapeDtypeStruct((1, nout), jnp.float32),
        scratch_shapes=[pltpu.VMEM((1, HID), jnp.float32)],
    )(aggp, z2, bl2, recip, Wf1, bf1, Wf2, bf2)


def _sc_agg(y, src3d, dst3d, d):
    mesh = plsc.VectorSubcoreMesh(core_axis_name="c", subcore_axis_name="s",
                                  num_cores=_NC, num_subcores=_NS)

    @functools.partial(
        pl.kernel,
        out_type=jax.ShapeDtypeStruct((_NC, _NPAD, d), jnp.float32),
        mesh=mesh,
        scratch_types=[
            pltpu.VMEM((_NCHUNK, _CH), jnp.int32),
            pltpu.VMEM((_NCHUNK, _CH), jnp.int32),
            pltpu.VMEM((_CH, d), jnp.float32),
            pltpu.VMEM((_RCOPY, d), jnp.float32),
            pltpu.VMEM_SHARED((_NPAD, d), jnp.float32),
            pltpu.SemaphoreType.DMA,
        ],
        compiler_params=pltpu.CompilerParams(use_tc_tiling_on_sc=False),
    )
    def k(y_hbm, src_hbm, dst_hbm, out_hbm, src_v, dst_v, rows_v, buf_v,
          agg_sh, sem):
        c = lax.axis_index("c")
        s = lax.axis_index("s")
        wid = s * _NC + c

        def zrow(i, _):
            for jj in range(d // 16):
                buf_v[i, pl.ds(jj * 16, 16)] = jnp.zeros((16,), jnp.float32)
            return 0
        lax.fori_loop(0, _RCOPY, zrow, 0)

        for r in range(_RPT // _RCOPY):
            pltpu.sync_copy(
                buf_v, agg_sh.at[pl.ds(s * _RPT + r * _RCOPY, _RCOPY)])

        pltpu.sync_copy(src_hbm.at[wid], src_v)
        pltpu.sync_copy(dst_hbm.at[wid], dst_v)

        plsc.subcore_barrier()

        def body(j, _):
            pltpu.async_copy(y_hbm.at[src_v.at[j]], rows_v, sem).wait()
            pltpu.sync_copy(rows_v, agg_sh.at[dst_v.at[j]], add=True)
            return 0
        lax.fori_loop(0, _NCHUNK, body, 0)

        plsc.subcore_barrier()

        for r in range(_RPT // _RCOPY):
            base = s * _RPT + r * _RCOPY
            pltpu.sync_copy(agg_sh.at[pl.ds(base, _RCOPY)], buf_v)
            pltpu.sync_copy(buf_v, out_hbm.at[c, pl.ds(base, _RCOPY)])

    return k(y, src3d, dst3d)


def kernel(x, edge_index, Wl1, bl1, Wr1, Wl2, bl2, Wr2, Wf1, bf1, Wf2, bf2):
    src3d = edge_index[0].reshape(_NW, _NCHUNK, _CH)
    dst3d = edge_index[1].reshape(_NW, _NCHUNK, _CH)

    y1, z1 = _tc1(x, Wl1, Wr1)
    aggp1 = _sc_agg(y1, src3d, dst3d, EXT)
    y2, z2, recip = _tc2(aggp1, z1, bl1.reshape(1, HID), Wl2, Wr2)
    aggp2 = _sc_agg(y2, src3d, dst3d, HID)
    return _tc3(aggp2, z2, bl2.reshape(1, HID), recip,
                Wf1, bf1.reshape(1, HID), Wf2, bf2.reshape(1, Wf2.shape[0]))


# speedup vs baseline: 12.6528x; 1.3519x over previous
import functools

import jax
import jax.numpy as jnp
from jax import lax
from jax.experimental import pallas as pl
from jax.experimental.pallas import tpu as pltpu
from jax.experimental.pallas import tpu_sc as plsc

N_NODES = 10000
N_EDGES = 320000
D_FEAT = 128
HID = 64
EXT = HID + 16

_NC = 2
_NS = 16
_NW = _NC * _NS

_CH = 80
_EPW = N_EDGES // _NW
_NCHUNK = _EPW // _CH
_NPAD = 10240
_RPT = _NPAD // _NS
_RCOPY = 128

_BLK = 2000
_GRID = N_NODES // _BLK


def _dot_t(a, w):
    return lax.dot_general(a, w, (((1,), (1,)), ((), ())),
                           preferred_element_type=jnp.float32)


def _tc1_body(x_ref, wl_ref, wr_ref, y_ref, z_ref):
    xb = x_ref[...]
    y = _dot_t(xb, wl_ref[...])
    z = _dot_t(xb, wr_ref[...])
    y_ref[...] = jnp.concatenate(
        [y, jnp.ones((y.shape[0], EXT - HID), jnp.float32)], axis=1)
    z_ref[...] = z


def _tc1(x, Wl1, Wr1):
    return pl.pallas_call(
        _tc1_body,
        grid=(_GRID,),
        in_specs=[
            pl.BlockSpec((_BLK, D_FEAT), lambda i: (i, 0)),
            pl.BlockSpec((HID, D_FEAT), lambda i: (0, 0)),
            pl.BlockSpec((HID, D_FEAT), lambda i: (0, 0)),
        ],
        out_specs=[
            pl.BlockSpec((_BLK, EXT), lambda i: (i, 0)),
            pl.BlockSpec((_BLK, HID), lambda i: (i, 0)),
        ],
        out_shape=[
            jax.ShapeDtypeStruct((N_NODES, EXT), jnp.float32),
            jax.ShapeDtypeStruct((N_NODES, HID), jnp.float32),
        ],
    )(x, Wl1, Wr1)


def _tc2_body(aggp_ref, z1_ref, bl1_ref, wl2_ref, wr2_ref,
              y2_ref, z2_ref, recip_ref):
    a = aggp_ref[0] + aggp_ref[1]
    cnt = jnp.max(a[:, HID:EXT], axis=1, keepdims=True)
    recip = 1.0 / jnp.maximum(cnt, 1.0)
    h1 = jnp.maximum(a[:, :HID] * recip + bl1_ref[...] + z1_ref[...], 0.0)
    y2_ref[...] = _dot_t(h1, wl2_ref[...])
    z2_ref[...] = _dot_t(h1, wr2_ref[...])
    recip_ref[...] = recip


def _tc2(aggp, z1, bl1, Wl2, Wr2):
    return pl.pallas_call(
        _tc2_body,
        grid=(_GRID,),
        in_specs=[
            pl.BlockSpec((_NC, _BLK, EXT), lambda i: (0, i, 0)),
            pl.BlockSpec((_BLK, HID), lambda i: (i, 0)),
            pl.BlockSpec((1, HID), lambda i: (0, 0)),
            pl.BlockSpec((HID, HID), lambda i: (0, 0)),
            pl.BlockSpec((HID, HID), lambda i: (0, 0)),
        ],
        out_specs=[
            pl.BlockSpec((_BLK, HID), lambda i: (i, 0)),
            pl.BlockSpec((_BLK, HID), lambda i: (i, 0)),
            pl.BlockSpec((_BLK, 1), lambda i: (i, 0)),
        ],
        out_shape=[
            jax.ShapeDtypeStruct((N_NODES, HID), jnp.float32),
            jax.ShapeDtypeStruct((N_NODES, HID), jnp.float32),
            jax.ShapeDtypeStruct((N_NODES, 1), jnp.float32),
        ],
    )(aggp, z1, bl1, Wl2, Wr2)


def _tc3_body(aggp_ref, z2_ref, bl2_ref, recip_ref,
              wf1_ref, bf1_ref, wf2_ref, bf2_ref, out_ref, acc_ref):
    i = pl.program_id(0)

    @pl.when(i == 0)
    def _():
        acc_ref[...] = jnp.zeros_like(acc_ref)

    a = aggp_ref[0] + aggp_ref[1]
    h2 = jnp.maximum(a * recip_ref[...] + bl2_ref[...] + z2_ref[...], 0.0)
    acc_ref[...] += jnp.sum(h2, axis=0, keepdims=True)

    @pl.when(i == pl.num_programs(0) - 1)
    def _():
        g = acc_ref[...] * (1.0 / N_NODES)
        h = jnp.maximum(_dot_t(g, wf1_ref[...]) + bf1_ref[...], 0.0)
        o = _dot_t(h, wf2_ref[...]) + bf2_ref[...]
        m = jnp.max(o, axis=1, keepdims=True)
        e = jnp.exp(o - m)
        out_ref[...] = e / jnp.sum(e, axis=1, keepdims=True)


def _tc3(aggp, z2, bl2, recip, Wf1, bf1, Wf2, bf2):
    nout = Wf2.shape[0]
    return pl.pallas_call(
        _tc3_body,
        grid=(_GRID,),
        in_specs=[
            pl.BlockSpec((_NC, _BLK, HID), lambda i: (0, i, 0)),
            pl.BlockSpec((_BLK, HID), lambda i: (i, 0)),
            pl.BlockSpec((1, HID), lambda i: (0, 0)),
            pl.BlockSpec((_BLK, 1), lambda i: (i, 0)),
            pl.BlockSpec((HID, HID), lambda i: (0, 0)),
            pl.BlockSpec((1, HID), lambda i: (0, 0)),
            pl.BlockSpec((nout, HID), lambda i: (0, 0)),
            pl.BlockSpec((1, nout), lambda i: (0, 0)),
        ],
        out_specs=pl.BlockSpec((1, nout), lambda i: (0, 0)),
        out_shape=jax.ShapeDtypeStruct((1, nout), jnp.float32),
        scratch_shapes=[pltpu.VMEM((1, HID), jnp.float32)],
    )(aggp, z2, bl2, recip, Wf1, bf1, Wf2, bf2)


def _sc_agg(y, src3d, dst3d, d, stage_y):
    mesh = plsc.VectorSubcoreMesh(core_axis_name="c", subcore_axis_name="s",
                                  num_cores=_NC, num_subcores=_NS)

    @functools.partial(
        pl.kernel,
        out_type=jax.ShapeDtypeStruct((_NC, _NPAD, d), jnp.float32),
        mesh=mesh,
        scratch_types=[
            pltpu.VMEM((_NCHUNK, _CH), jnp.int32),
            pltpu.VMEM((_NCHUNK, _CH), jnp.int32),
            pltpu.VMEM((_CH, d), jnp.float32),
            pltpu.VMEM((_CH, d), jnp.float32),
            pltpu.VMEM((_RCOPY, d), jnp.float32),
            pltpu.VMEM_SHARED((N_NODES if stage_y else 1, d), jnp.float32),
            pltpu.VMEM_SHARED((_NPAD, d), jnp.float32),
            pltpu.SemaphoreType.DMA,
            pltpu.SemaphoreType.DMA,
        ],
        compiler_params=pltpu.CompilerParams(use_tc_tiling_on_sc=False),
    )
    def k(y_hbm, src_hbm, dst_hbm, out_hbm, src_v, dst_v, rows0_v, rows1_v,
          buf_v, y_sh, agg_sh, sem0, sem1):
        c = lax.axis_index("c")
        s = lax.axis_index("s")
        wid = s * _NC + c

        if stage_y:
            @pl.when(s == 0)
            def _():
                pltpu.async_copy(y_hbm, y_sh, sem1)

        def zrow(i, _):
            for jj in range(d // 16):
                buf_v[i, pl.ds(jj * 16, 16)] = jnp.zeros((16,), jnp.float32)
            return 0
        lax.fori_loop(0, _RCOPY, zrow, 0)

        for r in range(_RPT // _RCOPY):
            pltpu.sync_copy(
                buf_v, agg_sh.at[pl.ds(s * _RPT + r * _RCOPY, _RCOPY)])

        pltpu.sync_copy(src_hbm.at[wid], src_v)
        pltpu.sync_copy(dst_hbm.at[wid], dst_v)

        if stage_y:
            @pl.when(s == 0)
            def _():
                pltpu.make_async_copy(y_hbm, y_sh, sem1).wait()

        plsc.subcore_barrier()

        y_src = y_sh if stage_y else y_hbm

        def gather(j, buf, sem):
            return pltpu.async_copy(y_src.at[src_v.at[j]], buf, sem)

        def gwait(j, buf, sem):
            pltpu.make_async_copy(y_src.at[src_v.at[j]], buf, sem).wait()

        def scat(j, buf):
            pltpu.sync_copy(buf, agg_sh.at[dst_v.at[j]], add=True)

        gather(0, rows0_v, sem0)

        def body(i, _):
            a = 2 * i
            gwait(a, rows0_v, sem0)
            gather(a + 1, rows1_v, sem1)
            scat(a, rows0_v)
            gwait(a + 1, rows1_v, sem1)
            gather(a + 2, rows0_v, sem0)
            scat(a + 1, rows1_v)
            return 0
        lax.fori_loop(0, (_NCHUNK - 1) // 2, body, 0)

        gwait(_NCHUNK - 1, rows0_v, sem0)
        scat(_NCHUNK - 1, rows0_v)

        plsc.subcore_barrier()

        for r in range(_RPT // _RCOPY):
            base = s * _RPT + r * _RCOPY
            pltpu.sync_copy(agg_sh.at[pl.ds(base, _RCOPY)], buf_v)
            pltpu.sync_copy(buf_v, out_hbm.at[c, pl.ds(base, _RCOPY)])

    return k(y, src3d, dst3d)


def kernel(x, edge_index, Wl1, bl1, Wr1, Wl2, bl2, Wr2, Wf1, bf1, Wf2, bf2):
    src3d = edge_index[0].reshape(_NW, _NCHUNK, _CH)
    dst3d = edge_index[1].reshape(_NW, _NCHUNK, _CH)

    y1, z1 = _tc1(x, Wl1, Wr1)
    aggp1 = _sc_agg(y1, src3d, dst3d, EXT, stage_y=False)
    y2, z2, recip = _tc2(aggp1, z1, bl1.reshape(1, HID), Wl2, Wr2)
    aggp2 = _sc_agg(y2, src3d, dst3d, HID, stage_y=True)
    return _tc3(aggp2, z2, bl2.reshape(1, HID), recip,
                Wf1, bf1.reshape(1, HID), Wf2, bf2.reshape(1, Wf2.shape[0]))
